# baseline (speedup 1.0000x reference)
import functools

import jax
import jax.numpy as jnp
from jax import lax
from jax.experimental import pallas as pl
from jax.experimental.pallas import tpu as pltpu
from jax.experimental.pallas import tpu_sc as plsc

_E = 8
_TOPK = 2
_BR = 4096
_NC = 2
_NS = 16
_L = 16


def _gate_block(x_ref, w_ref, b_ref, logits_ref):
    x = x_ref[...]
    w = w_ref[...]
    logits = jax.lax.dot_general(
        x, w, (((1,), (1,)), ((), ())), preferred_element_type=jnp.float32
    ) + b_ref[...]
    logits_ref[...] = logits.T


def _make_router(n_tokens):
    nw = _NC * _NS
    tpw = n_tokens // nw
    mesh = plsc.VectorSubcoreMesh(
        core_axis_name="c", subcore_axis_name="s", num_cores=_NC
    )

    @functools.partial(
        pl.kernel,
        mesh=mesh,
        out_type=[
            jax.ShapeDtypeStruct((_TOPK, n_tokens), jnp.float32),
            jax.ShapeDtypeStruct((_TOPK, n_tokens), jnp.int32),
        ],
        scratch_types=[
            pltpu.VMEM((_E, tpw), jnp.float32),
            pltpu.VMEM((_TOPK, tpw), jnp.float32),
            pltpu.VMEM((_TOPK, tpw), jnp.int32),
        ],
    )
    def _route(logits_hbm, vals_hbm, idx_hbm, lg_v, vals_v, idx_v):
        wid = lax.axis_index("s") * _NC + lax.axis_index("c")
        base = wid * tpw
        pltpu.sync_copy(logits_hbm.at[:, pl.ds(base, tpw)], lg_v)

        neg_inf = jnp.full((_L,), -jnp.inf, jnp.float32)

        def step(j, carry):
            s = pl.ds(j * _L, _L)
            rows = [lg_v[e, s] for e in range(_E)]
            m1 = rows[0]
            for e in range(1, _E):
                m1 = jnp.maximum(m1, rows[e])
            idx1 = jnp.full((_L,), _E - 1, jnp.int32)
            for e in range(_E - 2, -1, -1):
                idx1 = jnp.where(rows[e] == m1, e, idx1)
            m2 = neg_inf
            for e in range(_E):
                m2 = jnp.maximum(m2, jnp.where(idx1 == e, neg_inf, rows[e]))
            idx2 = jnp.full((_L,), 0, jnp.int32)
            for e in range(_E - 1, -1, -1):
                idx2 = jnp.where((rows[e] == m2) & (idx1 != e), e, idx2)
            w1 = 1.0 / (1.0 + jnp.exp(m2 - m1))
            vals_v[0, s] = w1
            vals_v[1, s] = 1.0 - w1
            idx_v[0, s] = idx1
            idx_v[1, s] = idx2
            return carry

        lax.fori_loop(0, tpw // _L, step, 0)
        pltpu.sync_copy(vals_v, vals_hbm.at[:, pl.ds(base, tpw)])
        pltpu.sync_copy(idx_v, idx_hbm.at[:, pl.ds(base, tpw)])

    return _route


def kernel(hidden_states, W, b):
    orig_shape = hidden_states.shape
    x = hidden_states.reshape(-1, orig_shape[-1])
    n_tokens, hidden = x.shape
    grid = (n_tokens // _BR,)

    logits_t = pl.pallas_call(
        _gate_block,
        grid=grid,
        in_specs=[
            pl.BlockSpec((_BR, hidden), lambda i: (i, 0)),
            pl.BlockSpec((_E, hidden), lambda i: (0, 0)),
            pl.BlockSpec((1, _E), lambda i: (0, 0)),
        ],
        out_specs=[pl.BlockSpec((_E, _BR), lambda i: (0, i))],
        out_shape=[jax.ShapeDtypeStruct((_E, n_tokens), jnp.float32)],
        compiler_params=pltpu.CompilerParams(
            dimension_semantics=("arbitrary",),
        ),
    )(x, W, b.reshape(1, _E))[0]

    vals_t, idx_t = _make_router(n_tokens)(logits_t)
    return (logits_t.T, vals_t.T, idx_t.T)


# speedup vs baseline: 1.3420x; 1.3420x over previous
import jax
import jax.numpy as jnp
from jax.experimental import pallas as pl
from jax.experimental.pallas import tpu as pltpu

_E = 8
_TOPK = 2
_BR = 4096


def _router_block(x_ref, w_ref, b_ref, logits_ref, vals_ref, idx_ref):
    x = x_ref[...]
    w = w_ref[...]
    logits = jax.lax.dot_general(
        x, w, (((1,), (1,)), ((), ())), preferred_element_type=jnp.float32
    ) + b_ref[...]
    logits_ref[...] = logits.T

    m1 = jnp.max(logits, axis=-1, keepdims=True)
    i1 = jnp.argmax(logits, axis=-1)
    iota = jax.lax.broadcasted_iota(jnp.int32, logits.shape, 1)
    masked = jnp.where(iota == i1[:, None], -jnp.inf, logits)
    m2 = jnp.max(masked, axis=-1, keepdims=True)
    i2 = jnp.argmax(masked, axis=-1)
    w1 = 1.0 / (1.0 + jnp.exp(m2 - m1))
    vals_ref[...] = jnp.concatenate([w1.T, 1.0 - w1.T], axis=0)
    idx_ref[...] = jnp.concatenate([i1[None, :], i2[None, :]], axis=0)


def kernel(hidden_states, W, b):
    orig_shape = hidden_states.shape
    x = hidden_states.reshape(-1, orig_shape[-1])
    n_tokens, hidden = x.shape
    grid = (n_tokens // _BR,)

    logits_t, vals_t, idx_t = pl.pallas_call(
        _router_block,
        grid=grid,
        in_specs=[
            pl.BlockSpec((_BR, hidden), lambda i: (i, 0)),
            pl.BlockSpec((_E, hidden), lambda i: (0, 0)),
            pl.BlockSpec((1, _E), lambda i: (0, 0)),
        ],
        out_specs=[
            pl.BlockSpec((_E, _BR), lambda i: (0, i)),
            pl.BlockSpec((_TOPK, _BR), lambda i: (0, i)),
            pl.BlockSpec((_TOPK, _BR), lambda i: (0, i)),
        ],
        out_shape=[
            jax.ShapeDtypeStruct((_E, n_tokens), jnp.float32),
            jax.ShapeDtypeStruct((_TOPK, n_tokens), jnp.float32),
            jax.ShapeDtypeStruct((_TOPK, n_tokens), jnp.int32),
        ],
        compiler_params=pltpu.CompilerParams(
            dimension_semantics=("arbitrary",),
        ),
    )(x, W, b.reshape(1, _E))

    return (logits_t.T, vals_t.T, idx_t.T)
